# Initial kernel scaffold; baseline (speedup 1.0000x reference)
import jax
import jax.numpy as jnp
from jax import lax
from jax.experimental import pallas as pl
from jax.experimental.pallas import tpu as pltpu
from jax.experimental.pallas import tpu_sc as plsc

NUM_ACTIONS = 16
D = 128
NUM_ARGS = 3
L_TOTAL = 819200

NC = 2
NS = 16
NW = NC * NS
CHUNK = 512
GATHER = 128


def _build_combined(table):
    def body(t_ref, out_ref):
        t = t_ref[...]
        tpad = jnp.concatenate(
            [t, jnp.zeros((NUM_ACTIONS, 1), jnp.float32)], axis=1)
        row = lax.broadcasted_iota(jnp.int32, (NUM_ACTIONS * NUM_ARGS, NUM_ACTIONS), 0)
        col = lax.broadcasted_iota(jnp.int32, (NUM_ACTIONS * NUM_ARGS, NUM_ACTIONS), 1)
        onehot = (row // NUM_ARGS == col).astype(jnp.float32)
        comb = jnp.dot(onehot, tpad, preferred_element_type=jnp.float32)
        g = lax.broadcasted_iota(jnp.int32, (NUM_ACTIONS * NUM_ARGS, D), 0) % NUM_ARGS
        is_last = lax.broadcasted_iota(jnp.int32, (NUM_ACTIONS * NUM_ARGS, D), 1) == D - 1
        out_ref[...] = comb + jnp.where(is_last, g.astype(jnp.float32), 0.0)

    return pl.pallas_call(
        body,
        out_shape=jax.ShapeDtypeStruct((NUM_ACTIONS * NUM_ARGS, D), jnp.float32),
    )(table)


def _sc_lookup_body(comb_hbm, act_hbm, arg_hbm, out_hbm, a_v, g_v, idx_v, rows_v, sem):
    rows_per_w = L_TOTAL // NW
    wid = lax.axis_index("s") * NC + lax.axis_index("c")
    w_base = wid * rows_per_w

    def chunk_body(ci, carry):
        base = w_base + ci * CHUNK
        pltpu.sync_copy(act_hbm.at[pl.ds(base, CHUNK)], a_v)
        pltpu.sync_copy(arg_hbm.at[pl.ds(base, CHUNK)], g_v)
        for i in range(CHUNK // 16):
            a = a_v[pl.ds(i * 16, 16)]
            g = g_v[pl.ds(i * 16, 16)]
            idx_v[i // 8, pl.ds((i % 8) * 16, 16)] = a * NUM_ARGS + g
        copies = []
        for j in range(CHUNK // GATHER):
            copies.append(
                pltpu.async_copy(
                    comb_hbm.at[idx_v.at[j]],
                    rows_v.at[pl.ds(j * GATHER, GATHER)],
                    sem,
                ))
        for cp in copies:
            cp.wait()
        pltpu.sync_copy(rows_v, out_hbm.at[pl.ds(base, CHUNK)])
        return carry

    lax.fori_loop(0, rows_per_w // CHUNK, chunk_body, 0, unroll=False)


@jax.jit
def kernel(actions, arguments, table):
    comb = _build_combined(table.astype(jnp.float32))
    act = actions.astype(jnp.int32)
    arg = arguments.astype(jnp.int32)

    mesh = plsc.VectorSubcoreMesh(core_axis_name="c", subcore_axis_name="s")
    lookup = pl.kernel(
        _sc_lookup_body,
        out_type=jax.ShapeDtypeStruct((L_TOTAL, D), jnp.float32),
        mesh=mesh,
        scratch_types=[
            pltpu.VMEM((CHUNK,), jnp.int32),
            pltpu.VMEM((CHUNK,), jnp.int32),
            pltpu.VMEM((CHUNK // GATHER, GATHER), jnp.int32),
            pltpu.VMEM((CHUNK, D), jnp.float32),
            pltpu.SemaphoreType.DMA,
        ],
    )
    return lookup(comb, act, arg)


# speedup vs baseline: 2.8685x; 2.8685x over previous
import jax
import jax.numpy as jnp
from jax import lax
from jax.experimental import pallas as pl
from jax.experimental.pallas import tpu as pltpu
from jax.experimental.pallas import tpu_sc as plsc

NUM_ACTIONS = 16
D = 128
NUM_ARGS = 3
L_TOTAL = 819200

NC = 2
NS = 16
NW = NC * NS
CHUNK = 512
GATHER = 128


def _build_combined(table):
    def body(t_ref, out_ref):
        t = t_ref[...]
        tb = jnp.broadcast_to(t[None], (NUM_ARGS, NUM_ACTIONS, D - 1))
        g = lax.broadcasted_iota(jnp.int32, (NUM_ARGS, NUM_ACTIONS, 1), 0).astype(jnp.float32)
        comb = jnp.concatenate([tb, g], axis=2)
        out_ref[...] = comb.reshape(NUM_ARGS * NUM_ACTIONS, D)

    return pl.pallas_call(
        body,
        out_shape=jax.ShapeDtypeStruct((NUM_ACTIONS * NUM_ARGS, D), jnp.float32),
    )(table)


def _sc_lookup_body(comb_hbm, act_hbm, arg_hbm, out_hbm, a_v, g_v, idx_v, rows_v, sem):
    rows_per_w = L_TOTAL // NW
    wid = lax.axis_index("s") * NC + lax.axis_index("c")
    w_base = wid * rows_per_w

    def chunk_body(ci, carry):
        base = w_base + ci * CHUNK
        pltpu.sync_copy(act_hbm.at[pl.ds(base, CHUNK)], a_v)
        pltpu.sync_copy(arg_hbm.at[pl.ds(base, CHUNK)], g_v)
        for i in range(CHUNK // 16):
            a = a_v[pl.ds(i * 16, 16)]
            g = g_v[pl.ds(i * 16, 16)]
            idx_v[i // 8, pl.ds((i % 8) * 16, 16)] = g * NUM_ACTIONS + a
        copies = []
        for j in range(CHUNK // GATHER):
            copies.append(
                pltpu.async_copy(
                    comb_hbm.at[idx_v.at[j]],
                    rows_v.at[pl.ds(j * GATHER, GATHER)],
                    sem,
                ))
        for cp in copies:
            cp.wait()
        pltpu.sync_copy(rows_v, out_hbm.at[pl.ds(base, CHUNK)])
        return carry

    lax.fori_loop(0, rows_per_w // CHUNK, chunk_body, 0, unroll=False)


@jax.jit
def kernel(actions, arguments, table):
    comb = _build_combined(table.astype(jnp.float32))
    act = actions.astype(jnp.int32)
    arg = arguments.astype(jnp.int32)

    mesh = plsc.VectorSubcoreMesh(core_axis_name="c", subcore_axis_name="s")
    lookup = pl.kernel(
        _sc_lookup_body,
        out_type=jax.ShapeDtypeStruct((L_TOTAL, D), jnp.float32),
        mesh=mesh,
        scratch_types=[
            pltpu.VMEM((CHUNK,), jnp.int32),
            pltpu.VMEM((CHUNK,), jnp.int32),
            pltpu.VMEM((CHUNK // GATHER, GATHER), jnp.int32),
            pltpu.VMEM((CHUNK, D), jnp.float32),
            pltpu.SemaphoreType.DMA,
        ],
    )
    return lookup(comb, act, arg)
